# speedup vs baseline: 3.5915x; 3.5915x over previous
import jax
import jax.numpy as jnp
from jax.experimental import pallas as pl
from jax.experimental.pallas import tpu as pltpu

B = 16
T = 32768
BAG = 128
CLS = 10
FEAT_ROOT = 192
MASK_ROOT = 2
FEAT_SUB = 128
MASK_SUB = 4

NB = 16
BT = T // NB


def _leaky(x):
    return jnp.where(x >= 0, x, 0.01 * x)


def _seg_count(t_idx, cu_ref):
    seg = jnp.zeros(t_idx.shape, jnp.int32)
    for s in range(1, B):
        seg += (t_idx >= cu_ref[s]).astype(jnp.int32)
    return seg


def _body(sub_f_ref, sub_m_ref, cu_ref, root_f_ref, root_m_ref,
          Wsub_ref, bsub_ref, gsub_ref, besub_ref,
          Wroot_ref, broot_ref, groot_ref, beroot_ref,
          Wv_ref, bv_ref, Wcp_ref, bcp_ref, Wca_ref, bca_ref,
          Wor_ref, bor_ref, Wos_ref, bos_ref,
          o_v_ref, o_cls_p_ref, root_cat_ref, out_sub_ref,
          acc_ref, r4_ref, part_ref):
    i = pl.program_id(0)

    @pl.when(i < NB)
    def _pass1():
        x = sub_f_ref[...]
        m = sub_m_ref[...].astype(jnp.float32)
        xs = (jnp.dot(x, Wsub_ref[0:FEAT_SUB, :], preferred_element_type=jnp.float32)
              + jnp.dot(m, Wsub_ref[FEAT_SUB:FEAT_SUB + MASK_SUB, :],
                        preferred_element_type=jnp.float32)
              + bsub_ref[...])
        x_raw = _leaky(xs)

        p = jnp.dot(x_raw, Wos_ref[BAG:2 * BAG, :],
                    preferred_element_type=jnp.float32)
        p = jnp.where(m >= 1.0, -jnp.inf, p)
        part_ref[pl.ds(i * BT, BT), :] = p

        t_row = jax.lax.broadcasted_iota(jnp.int32, (1, BT), 1) + i * BT
        seg_row = _seg_count(t_row, cu_ref)
        onehot = (jnp.broadcast_to(seg_row, (B, BT))
                  == jax.lax.broadcasted_iota(jnp.int32, (B, BT), 0)
                  ).astype(jnp.float32)
        contrib = jnp.dot(onehot, x_raw, preferred_element_type=jnp.float32)

        @pl.when(i == 0)
        def _():
            acc_ref[...] = contrib

        @pl.when(i > 0)
        def _():
            acc_ref[...] = acc_ref[...] + contrib

    @pl.when(i == NB)
    def _root():
        io = jax.lax.broadcasted_iota(jnp.int32, (B, 1), 0)
        denom = jnp.ones((B, 1), jnp.float32)
        nz = jnp.zeros((B, 1), jnp.float32)
        for s in range(B):
            l = cu_ref[s + 1] - cu_ref[s]
            denom = jnp.where(io == s, jnp.maximum(l, 1).astype(jnp.float32), denom)
            nz = jnp.where(io == s, (l > 0).astype(jnp.float32), nz)
        x_agg = acc_ref[...] / denom * nz

        mu = jnp.mean(x_agg, axis=1, keepdims=True)
        var = jnp.mean((x_agg - mu) ** 2, axis=1, keepdims=True)
        x_agg = (x_agg - mu) / jnp.sqrt(var + 1e-5) * gsub_ref[...] + besub_ref[...]

        rm = root_m_ref[...].astype(jnp.float32)
        xr = (jnp.dot(x_agg, Wroot_ref[0:BAG, :], preferred_element_type=jnp.float32)
              + jnp.dot(root_f_ref[:, BAG:FEAT_ROOT],
                        Wroot_ref[BAG:FEAT_ROOT, :], preferred_element_type=jnp.float32)
              + jnp.dot(rm, Wroot_ref[FEAT_ROOT:FEAT_ROOT + MASK_ROOT, :],
                        preferred_element_type=jnp.float32)
              + broot_ref[...])
        xr = _leaky(xr)
        mu2 = jnp.mean(xr, axis=1, keepdims=True)
        var2 = jnp.mean((xr - mu2) ** 2, axis=1, keepdims=True)
        res = (xr - mu2) / jnp.sqrt(var2 + 1e-5) * groot_ref[...] + beroot_ref[...]

        o_v_ref[...] = jnp.dot(res, Wv_ref[...],
                               preferred_element_type=jnp.float32) + bv_ref[...]
        o_cls_p_ref[...] = jnp.dot(res, Wcp_ref[...],
                                   preferred_element_type=jnp.float32) + bcp_ref[...]
        oca = jnp.dot(res, Wca_ref[...], preferred_element_type=jnp.float32) + bca_ref[...]
        orr = jnp.dot(res, Wor_ref[...], preferred_element_type=jnp.float32) + bor_ref[...]
        orr = jnp.where(rm >= 1.0, -jnp.inf, orr)
        root_cat_ref[...] = jnp.concatenate([oca, orr], axis=1)

        r4_ref[...] = jnp.dot(res, Wos_ref[0:BAG, :],
                              preferred_element_type=jnp.float32) + bos_ref[...]

    @pl.when(i >= NB)
    def _pass2():
        j = i - NB
        t_col = jax.lax.broadcasted_iota(jnp.int32, (BT, 1), 0) + j * BT
        seg = _seg_count(t_col, cu_ref)
        oneh = (jnp.broadcast_to(seg, (BT, B))
                == jax.lax.broadcasted_iota(jnp.int32, (BT, B), 1)
                ).astype(jnp.float32)
        g = jnp.dot(oneh, r4_ref[...], preferred_element_type=jnp.float32)
        out_sub_ref[...] = part_ref[pl.ds(j * BT, BT), :] + g


def kernel(root_feats, root_mask, sub_feats, sub_mask, cu_seqlens,
           W_sub, b_sub, g_sub, be_sub,
           W_root, b_root, g_root, be_root,
           W_v, b_v, W_cls_p, b_cls_p, W_cls_a, b_cls_a,
           W_out_root, b_out_root, W_out_sub, b_out_sub):
    cu = cu_seqlens.astype(jnp.int32).at[0].set(0).at[-1].set(T)

    row = lambda v: v.reshape(1, -1)

    first = lambda idx: (0, 0)
    tok = lambda idx: (jnp.minimum(idx, NB - 1), 0)

    out_shapes = (
        jax.ShapeDtypeStruct((B, 1), jnp.float32),
        jax.ShapeDtypeStruct((B, CLS), jnp.float32),
        jax.ShapeDtypeStruct((B, 1 + MASK_ROOT), jnp.float32),
        jax.ShapeDtypeStruct((T, MASK_SUB), jnp.float32),
    )
    out_specs = (
        pl.BlockSpec((B, 1), first),
        pl.BlockSpec((B, CLS), first),
        pl.BlockSpec((B, 1 + MASK_ROOT), first),
        pl.BlockSpec((BT, MASK_SUB), lambda idx: (jnp.maximum(idx - NB, 0), 0)),
    )
    weights = (W_sub, row(b_sub), row(g_sub), row(be_sub),
               W_root, row(b_root), row(g_root), row(be_root),
               W_v, row(b_v), W_cls_p, row(b_cls_p), W_cls_a, row(b_cls_a),
               W_out_root, row(b_out_root), W_out_sub, row(b_out_sub))
    in_specs = [
        pl.BlockSpec((BT, FEAT_SUB), tok),
        pl.BlockSpec((BT, MASK_SUB), tok),
        pl.BlockSpec(memory_space=pltpu.SMEM),
        pl.BlockSpec((B, FEAT_ROOT), first),
        pl.BlockSpec((B, MASK_ROOT), first),
    ] + [pl.BlockSpec(w.shape, first) for w in weights]

    o_v, o_cls_p, root_cat, out_sub = pl.pallas_call(
        _body,
        grid=(2 * NB,),
        in_specs=in_specs,
        out_specs=out_specs,
        out_shape=out_shapes,
        scratch_shapes=[
            pltpu.VMEM((B, BAG), jnp.float32),
            pltpu.VMEM((B, MASK_SUB), jnp.float32),
            pltpu.VMEM((T, MASK_SUB), jnp.float32),
        ],
        compiler_params=pltpu.CompilerParams(
            dimension_semantics=("arbitrary",),
        ),
    )(sub_feats, sub_mask, cu, root_feats, root_mask, *weights)

    return (o_v, o_cls_p, root_cat, out_sub)


# speedup vs baseline: 4.5159x; 1.2574x over previous
import jax
import jax.numpy as jnp
from jax.experimental import pallas as pl
from jax.experimental.pallas import tpu as pltpu

B = 16
T = 32768
BAG = 128
CLS = 10
FEAT_ROOT = 192
MASK_ROOT = 2
FEAT_SUB = 128
MASK_SUB = 4

NB = 16
BT = T // NB


def _leaky(x):
    return jnp.where(x >= 0, x, 0.01 * x)


def _seg_count(t_idx, cu_ref):
    seg = jnp.zeros(t_idx.shape, jnp.int32)
    for s in range(1, B):
        seg += (t_idx >= cu_ref[s]).astype(jnp.int32)
    return seg


def _body(sub_f_ref, sub_m_ref, cu_ref, root_f_ref, root_m_ref,
          Wsub_ref, bsub_ref, gsub_ref, besub_ref,
          Wroot_ref, broot_ref, groot_ref, beroot_ref,
          Wv_ref, bv_ref, Wcp_ref, bcp_ref, Wca_ref, bca_ref,
          Wor_ref, bor_ref, Wos_ref, bos_ref,
          o_v_ref, o_cls_p_ref, root_cat_ref, out_sub_ref,
          acc_ref, r4_ref, part_ref, oneh_ref):
    i = pl.program_id(0)

    @pl.when(i < NB)
    def _pass1():
        x = sub_f_ref[...]
        m = sub_m_ref[...].astype(jnp.float32)
        xs = (jnp.dot(x, Wsub_ref[0:FEAT_SUB, :], preferred_element_type=jnp.float32)
              + jnp.dot(m, Wsub_ref[FEAT_SUB:FEAT_SUB + MASK_SUB, :],
                        preferred_element_type=jnp.float32)
              + bsub_ref[...])
        x_raw = _leaky(xs)

        p = jnp.dot(x_raw, Wos_ref[BAG:2 * BAG, :],
                    preferred_element_type=jnp.float32)
        p = jnp.where(m >= 1.0, -jnp.inf, p)
        part_ref[pl.ds(i * BT, BT), :] = p

        t_row = jax.lax.broadcasted_iota(jnp.int32, (1, BT), 1) + i * BT
        seg_row = _seg_count(t_row, cu_ref)
        onehot = (jnp.broadcast_to(seg_row, (B, BT))
                  == jax.lax.broadcasted_iota(jnp.int32, (B, BT), 0)
                  ).astype(jnp.float32)
        oneh_ref[:, pl.ds(i * BT, BT)] = onehot
        contrib = jnp.dot(onehot, x_raw, preferred_element_type=jnp.float32)

        @pl.when(i == 0)
        def _():
            acc_ref[...] = contrib

        @pl.when(i > 0)
        def _():
            acc_ref[...] = acc_ref[...] + contrib

    @pl.when(i == NB)
    def _root():
        io = jax.lax.broadcasted_iota(jnp.int32, (B, 1), 0)
        denom = jnp.ones((B, 1), jnp.float32)
        nz = jnp.zeros((B, 1), jnp.float32)
        for s in range(B):
            l = cu_ref[s + 1] - cu_ref[s]
            denom = jnp.where(io == s, jnp.maximum(l, 1).astype(jnp.float32), denom)
            nz = jnp.where(io == s, (l > 0).astype(jnp.float32), nz)
        x_agg = acc_ref[...] / denom * nz

        mu = jnp.mean(x_agg, axis=1, keepdims=True)
        var = jnp.mean((x_agg - mu) ** 2, axis=1, keepdims=True)
        x_agg = (x_agg - mu) / jnp.sqrt(var + 1e-5) * gsub_ref[...] + besub_ref[...]

        rm = root_m_ref[...].astype(jnp.float32)
        xr = (jnp.dot(x_agg, Wroot_ref[0:BAG, :], preferred_element_type=jnp.float32)
              + jnp.dot(root_f_ref[:, BAG:FEAT_ROOT],
                        Wroot_ref[BAG:FEAT_ROOT, :], preferred_element_type=jnp.float32)
              + jnp.dot(rm, Wroot_ref[FEAT_ROOT:FEAT_ROOT + MASK_ROOT, :],
                        preferred_element_type=jnp.float32)
              + broot_ref[...])
        xr = _leaky(xr)
        mu2 = jnp.mean(xr, axis=1, keepdims=True)
        var2 = jnp.mean((xr - mu2) ** 2, axis=1, keepdims=True)
        res = (xr - mu2) / jnp.sqrt(var2 + 1e-5) * groot_ref[...] + beroot_ref[...]

        o_v_ref[...] = jnp.dot(res, Wv_ref[...],
                               preferred_element_type=jnp.float32) + bv_ref[...]
        o_cls_p_ref[...] = jnp.dot(res, Wcp_ref[...],
                                   preferred_element_type=jnp.float32) + bcp_ref[...]
        oca = jnp.dot(res, Wca_ref[...], preferred_element_type=jnp.float32) + bca_ref[...]
        orr = jnp.dot(res, Wor_ref[...], preferred_element_type=jnp.float32) + bor_ref[...]
        orr = jnp.where(rm >= 1.0, -jnp.inf, orr)
        root_cat_ref[...] = jnp.concatenate([oca, orr], axis=1)

        r4_ref[...] = jnp.dot(res, Wos_ref[0:BAG, :],
                              preferred_element_type=jnp.float32) + bos_ref[...]

    @pl.when(i >= NB)
    def _pass2():
        j = i - NB
        oneh_row = oneh_ref[:, pl.ds(j * BT, BT)]
        g = jax.lax.dot_general(oneh_row, r4_ref[...],
                                (((0,), (0,)), ((), ())),
                                preferred_element_type=jnp.float32)
        out_sub_ref[...] = part_ref[pl.ds(j * BT, BT), :] + g


def kernel(root_feats, root_mask, sub_feats, sub_mask, cu_seqlens,
           W_sub, b_sub, g_sub, be_sub,
           W_root, b_root, g_root, be_root,
           W_v, b_v, W_cls_p, b_cls_p, W_cls_a, b_cls_a,
           W_out_root, b_out_root, W_out_sub, b_out_sub):
    cu = cu_seqlens.astype(jnp.int32).at[0].set(0).at[-1].set(T)

    row = lambda v: v.reshape(1, -1)

    first = lambda idx: (0, 0)
    tok = lambda idx: (jnp.minimum(idx, NB - 1), 0)

    out_shapes = (
        jax.ShapeDtypeStruct((B, 1), jnp.float32),
        jax.ShapeDtypeStruct((B, CLS), jnp.float32),
        jax.ShapeDtypeStruct((B, 1 + MASK_ROOT), jnp.float32),
        jax.ShapeDtypeStruct((T, MASK_SUB), jnp.float32),
    )
    out_specs = (
        pl.BlockSpec((B, 1), first),
        pl.BlockSpec((B, CLS), first),
        pl.BlockSpec((B, 1 + MASK_ROOT), first),
        pl.BlockSpec((BT, MASK_SUB), lambda idx: (jnp.maximum(idx - NB, 0), 0)),
    )
    weights = (W_sub, row(b_sub), row(g_sub), row(be_sub),
               W_root, row(b_root), row(g_root), row(be_root),
               W_v, row(b_v), W_cls_p, row(b_cls_p), W_cls_a, row(b_cls_a),
               W_out_root, row(b_out_root), W_out_sub, row(b_out_sub))
    in_specs = [
        pl.BlockSpec((BT, FEAT_SUB), tok),
        pl.BlockSpec((BT, MASK_SUB), tok),
        pl.BlockSpec(memory_space=pltpu.SMEM),
        pl.BlockSpec((B, FEAT_ROOT), first),
        pl.BlockSpec((B, MASK_ROOT), first),
    ] + [pl.BlockSpec(w.shape, first) for w in weights]

    o_v, o_cls_p, root_cat, out_sub = pl.pallas_call(
        _body,
        grid=(2 * NB,),
        in_specs=in_specs,
        out_specs=out_specs,
        out_shape=out_shapes,
        scratch_shapes=[
            pltpu.VMEM((B, BAG), jnp.float32),
            pltpu.VMEM((B, MASK_SUB), jnp.float32),
            pltpu.VMEM((T, MASK_SUB), jnp.float32),
            pltpu.VMEM((B, T), jnp.float32),
        ],
        compiler_params=pltpu.CompilerParams(
            dimension_semantics=("arbitrary",),
        ),
    )(sub_feats, sub_mask, cu, root_feats, root_mask, *weights)

    return (o_v, o_cls_p, root_cat, out_sub)
